# baseline (speedup 1.0000x reference)
import functools

import jax
import jax.numpy as jnp
import numpy as np
from jax import lax
from jax.experimental import pallas as pl
from jax.experimental.pallas import tpu as pltpu
from jax.experimental.pallas import tpu_sc as plsc

NUM_ITERATIONS = 10
ALPHA = 0.9
BETA = 0.9
GAMMA = 0.02
MAX_BFS_HOPS = 10
INF = 1e9

NC = 2
NS = 16
NW = NC * NS
K = 128
RB = 64
CS = 16


def _mesh():
    return plsc.VectorSubcoreMesh(core_axis_name="c", subcore_axis_name="s")


def _round_up(v, m):
    return (v + m - 1) // m * m


def _inj_constants(n, k_low):
    rng = np.random.RandomState(0)
    idxs, vals = [], []
    for _ in range(k_low):
        idxs.append(int(rng.choice(n, 1, replace=False)[0]))
        vals.append(float(rng.rand()))
    return idxs, vals


@functools.cache
def _make_scatter(C, nchunks, npad, acc_rows):
    acc_chunks = acc_rows // K
    per_tile_z = npad // NS

    @functools.partial(
        pl.kernel,
        mesh=_mesh(),
        compiler_params=pltpu.CompilerParams(use_tc_tiling_on_sc=False),
        out_type=[jax.ShapeDtypeStruct((NC, npad, C), jnp.float32)],
        scratch_types=[
            pltpu.VMEM_SHARED((acc_rows, C), jnp.float32),
            pltpu.VMEM((K,), jnp.int32),
            pltpu.VMEM((K,), jnp.int32),
            pltpu.VMEM((K, C), jnp.float32),
            pltpu.SemaphoreType.DMA,
        ],
    )
    def k(z, esrc, edst, partial, acc, sbuf, dbuf, gbuf, sem):
        c = lax.axis_index("c")
        s = lax.axis_index("s")
        w = c * NS + s

        zero = jnp.zeros((16,), jnp.float32)

        def zrow(i, _):
            for j in range(C // 16):
                gbuf[i, pl.ds(j * 16, 16)] = zero
            return 0

        lax.fori_loop(0, K, zrow, 0)
        per_tile_chunks = _round_up(acc_chunks, NS) // NS
        for t in range(per_tile_chunks):
            ch = s * per_tile_chunks + t

            @pl.when(ch < acc_chunks)
            def _():
                pltpu.sync_copy(gbuf, acc.at[pl.ds(ch * K, K)])

        plsc.subcore_barrier()

        def chunk(g, _):
            pltpu.sync_copy(esrc.at[w, pl.ds(g * K, K)], sbuf)
            pltpu.sync_copy(edst.at[w, pl.ds(g * K, K)], dbuf)
            pltpu.async_copy(z.at[sbuf], gbuf, sem).wait()
            pltpu.sync_copy(gbuf, acc.at[dbuf], add=True)
            return 0

        lax.fori_loop(0, nchunks, chunk, 0)
        plsc.subcore_barrier()
        pltpu.sync_copy(
            acc.at[pl.ds(s * per_tile_z, per_tile_z)],
            partial.at[c, pl.ds(s * per_tile_z, per_tile_z)],
        )

    return k


@functools.cache
def _make_update(C, npad):
    per_tile = npad // NW
    nblocks = per_tile // RB

    @functools.partial(
        pl.kernel,
        mesh=_mesh(),
        compiler_params=pltpu.CompilerParams(use_tc_tiling_on_sc=False),
        out_type=[
            jax.ShapeDtypeStruct((npad, C), jnp.float32),
            jax.ShapeDtypeStruct((npad, C), jnp.float32),
        ],
        scratch_types=[
            pltpu.VMEM((RB, C), jnp.float32),
            pltpu.VMEM((RB, C), jnp.float32),
            pltpu.VMEM((RB, C), jnp.float32),
            pltpu.VMEM((RB, C), jnp.float32),
            pltpu.VMEM((RB, C), jnp.float32),
            pltpu.VMEM((RB, C), jnp.float32),
        ],
    )
    def k(partial, mx, invm, arow, z_new, out_new, p0b, p1b, mxb, ivb, arb, ob):
        c = lax.axis_index("c")
        s = lax.axis_index("s")
        w = c * NS + s
        base = w * per_tile
        for b in range(nblocks):
            r0 = base + b * RB
            pltpu.sync_copy(partial.at[0, pl.ds(r0, RB)], p0b)
            pltpu.sync_copy(partial.at[1, pl.ds(r0, RB)], p1b)
            pltpu.sync_copy(mx.at[pl.ds(r0, RB)], mxb)
            pltpu.sync_copy(invm.at[pl.ds(r0, RB)], ivb)
            pltpu.sync_copy(arow.at[pl.ds(r0, RB)], arb)

            def ub(i, _):
                for j in range(C // 16):
                    sl = pl.ds(j * 16, 16)
                    sv = p0b[i, sl] + p1b[i, sl]
                    ov = mxb[i, sl] + ivb[i, sl] * sv
                    ob[i, sl] = ov
                    arb[i, sl] = arb[i, sl] * ov
                return 0

            lax.fori_loop(0, RB, ub, 0)
            pltpu.sync_copy(ob, out_new.at[pl.ds(r0, RB)])
            pltpu.sync_copy(arb, z_new.at[pl.ds(r0, RB)])

    return k


@functools.cache
def _make_bfs_update(npad):
    per_tile = npad // NW
    nblocks = per_tile // RB

    @functools.partial(
        pl.kernel,
        mesh=_mesh(),
        compiler_params=pltpu.CompilerParams(use_tc_tiling_on_sc=False),
        out_type=[
            jax.ShapeDtypeStruct((npad, CS), jnp.float32),
            jax.ShapeDtypeStruct((npad, CS), jnp.float32),
            jax.ShapeDtypeStruct((NW, 16), jnp.float32),
        ],
        scratch_types=[
            pltpu.VMEM((RB, CS), jnp.float32),
            pltpu.VMEM((RB, CS), jnp.float32),
            pltpu.VMEM((RB, CS), jnp.float32),
            pltpu.VMEM((RB, CS), jnp.float32),
            pltpu.VMEM((16,), jnp.float32),
            pltpu.VMEM((16,), jnp.float32),
        ],
    )
    def k(partial, r, dist, hop, r_new, dist_new, changed,
          p0b, p1b, rb, db, hb, chb):
        c = lax.axis_index("c")
        s = lax.axis_index("s")
        w = c * NS + s
        base = w * per_tile
        pltpu.sync_copy(hop, hb)
        hv = hb[...]
        one = jnp.full((16,), 1.0, jnp.float32)
        zero = jnp.zeros((16,), jnp.float32)
        chacc = zero
        for b in range(nblocks):
            r0 = base + b * RB
            pltpu.sync_copy(partial.at[0, pl.ds(r0, RB)], p0b)
            pltpu.sync_copy(partial.at[1, pl.ds(r0, RB)], p1b)
            pltpu.sync_copy(r.at[pl.ds(r0, RB)], rb)
            pltpu.sync_copy(dist.at[pl.ds(r0, RB)], db)

            def ub(i, ch):
                cnt = p0b[i, :] + p1b[i, :]
                rv = rb[i, :]
                reached = (rv > 0.0) | (cnt > 0.0)
                rn = jnp.where(reached, one, zero)
                newly = reached & (rv <= 0.0)
                db[i, :] = jnp.where(newly, hv, db[i, :])
                rb[i, :] = rn
                return ch + jnp.where(newly, one, zero)

            chacc = lax.fori_loop(0, RB, ub, chacc)
            pltpu.sync_copy(rb, r_new.at[pl.ds(r0, RB)])
            pltpu.sync_copy(db, dist_new.at[pl.ds(r0, RB)])
        chb[...] = chacc
        pltpu.sync_copy(chb, changed.at[w])

    return k


@functools.cache
def _make_var(C, npad):
    per_tile = npad // NW
    nblocks = per_tile // RB
    nj = C // 16

    @functools.partial(
        pl.kernel,
        mesh=_mesh(),
        compiler_params=pltpu.CompilerParams(use_tc_tiling_on_sc=False),
        out_type=[jax.ShapeDtypeStruct((NW, 2 * C), jnp.float32)],
        scratch_types=[
            pltpu.VMEM((RB, C), jnp.float32),
            pltpu.VMEM((2 * C,), jnp.float32),
        ],
    )
    def k(xin, partials, xb, vb):
        c = lax.axis_index("c")
        s = lax.axis_index("s")
        w = c * NS + s
        base = w * per_tile
        zero = jnp.zeros((16,), jnp.float32)
        carry = tuple([zero] * (2 * nj))
        for b in range(nblocks):
            pltpu.sync_copy(xin.at[pl.ds(base + b * RB, RB)], xb)

            def ub(i, cr):
                out = []
                for j in range(nj):
                    v = xb[i, pl.ds(j * 16, 16)]
                    out.append(cr[j] + v)
                for j in range(nj):
                    v = xb[i, pl.ds(j * 16, 16)]
                    out.append(cr[nj + j] + v * v)
                return tuple(out)

            carry = lax.fori_loop(0, RB, ub, carry)
        for j in range(nj):
            vb[pl.ds(j * 16, 16)] = carry[j]
            vb[pl.ds(C + j * 16, 16)] = carry[nj + j]
        pltpu.sync_copy(vb, partials.at[w])

    return k


def _bfs(r0, dist0, bsrc, bdst, nchunks, npad, acc_rows):
    scat = _make_scatter(CS, nchunks, npad, acc_rows)
    upd = _make_bfs_update(npad)

    def cond(st):
        _, _, hop, chg = st
        return (hop <= float(MAX_BFS_HOPS)) & (chg > 0.0)

    def body(st):
        r, dist, hop, _ = st
        (part,) = scat(r, bsrc, bdst)
        rn, dn, ch = upd(part, r, dist, jnp.full((16,), hop, jnp.float32))
        return rn, dn, hop + 1.0, jnp.sum(ch)

    _, dist, _, _ = lax.while_loop(
        cond, body, (r0, dist0, jnp.float32(1.0), jnp.float32(1.0)))
    return jnp.minimum(dist, float(MAX_BFS_HOPS))


def kernel(x, edge_index, mask):
    n, d = x.shape
    e = edge_index.shape[1]
    k_low = int(d * GAMMA)
    inj_idx, inj_val = _inj_constants(n, k_low)

    npad = _round_up(n, NW * RB)
    ept = _round_up(-(-e // NW), K)
    epad = NW * ept
    acc_rows = _round_up(npad + 8, K)
    scratch_row = npad

    src = edge_index[0].astype(jnp.int32)
    dst = edge_index[1].astype(jnp.int32)
    maskf = mask.astype(jnp.float32)

    def pad_rows(a):
        return jnp.pad(a, ((0, npad - n), (0, 0)))

    def chans(cols):
        out = jnp.zeros((n, CS), jnp.float32)
        for i, cvec in enumerate(cols):
            out = out.at[:, i].set(cvec)
        return pad_rows(out)

    def edge_set(gidx, sidx):
        g = jnp.pad(gidx, (0, epad - e)).reshape(NW, ept)
        sc = jnp.pad(sidx, (0, epad - e),
                     constant_values=scratch_row).reshape(NW, ept)
        return g, sc

    fsrc, fdst = edge_set(src, dst)
    rsrc, rdst = edge_set(dst, src)
    bsrc = jnp.concatenate([fsrc, rsrc], axis=1)
    bdst = jnp.concatenate([fdst, rdst], axis=1)

    scat_s = _make_scatter(CS, ept // K, npad, acc_rows)
    scat_b = _make_scatter(d, ept // K, npad, acc_rows)
    upd_s = _make_update(CS, npad)
    upd_b = _make_update(d, npad)
    var_k = _make_var(d, npad)

    seeds = [maskf[:, 0]]
    for t in range(k_low):
        seeds.append(jnp.zeros((n,), jnp.float32).at[inj_idx[t]].set(1.0))
    r0 = chans(seeds)
    dist0 = jnp.where(r0 > 0, 0.0, INF) * jnp.pad(
        jnp.ones((n, CS), jnp.float32), ((0, npad - n), (0, 0)))
    distA = _bfs(r0, dist0, bsrc, bdst, 2 * ept // K, npad, acc_rows)
    f1 = distA[:n, 0]
    fmax = [distA[:n, 1 + t] for t in range(k_low)]

    a1 = ALPHA ** f1
    (part,) = scat_s(chans([a1]), fsrc, fdst)
    rowsum1 = (part[0] + part[1])[:n, 0]
    inv1 = 1.0 / (rowsum1 + 1e-12)
    mx1 = pad_rows(maskf * x)
    invm1 = pad_rows((1.0 - maskf) * inv1[:, None])
    arow1 = pad_rows(jnp.broadcast_to(a1[:, None], (n, d)))
    z = arow1 * mx1
    out = None
    for _ in range(NUM_ITERATIONS):
        (part,) = scat_b(z, fsrc, fdst)
        z, out = upd_b(part, mx1, invm1, arow1)

    (pvar,) = var_k(out)
    sums = jnp.sum(pvar[:, :d], axis=0)
    sqs = jnp.sum(pvar[:, d:], axis=0)
    mean = sums / n
    var = sqs / n - mean * mean
    _, low_idx = lax.top_k(-var, k_low)
    low_mask = jnp.zeros((d,), bool).at[low_idx].set(True)
    pre = jnp.argmin(low_mask)
    x2 = x
    mask2f = maskf
    for t in range(k_low):
        x2 = x2.at[inj_idx[t], low_idx[t]].set(jnp.float32(inj_val[t]))
        mask2f = mask2f.at[inj_idx[t], low_idx[t]].set(1.0)

    seedb = jnp.take(mask2f, pre, axis=1)
    r0b = chans([seedb])
    dist0b = jnp.where(r0b > 0, 0.0, INF) * jnp.pad(
        jnp.ones((n, CS), jnp.float32), ((0, npad - n), (0, 0)))
    distB = _bfs(r0b, dist0b, bsrc, bdst, 2 * ept // K, npad, acc_rows)
    f2 = distB[:n, 0]

    a2 = ALPHA ** f2
    pc = [a2 * (BETA ** fmax[t]) for t in range(k_low)]
    (part,) = scat_s(chans([a2] + pc), fsrc, fdst)
    rs = part[0] + part[1]
    inv_pre = 1.0 / (rs[:n, 0] + 1e-12)
    inv_w = [1.0 / (rs[:n, 1 + t] + 1e-12) for t in range(k_low)]

    xc = [x2[:, low_idx[t]] for t in range(k_low)]
    mc = [mask2f[:, low_idx[t]] for t in range(k_low)]
    mxc = chans([mc[t] * xc[t] for t in range(k_low)])
    invmc = chans([(1.0 - mc[t]) * inv_w[t] for t in range(k_low)])
    arowc = chans(pc)
    zc = arowc * mxc
    outc = None
    for _ in range(NUM_ITERATIONS):
        (part,) = scat_s(zc, fsrc, fdst)
        zc, outc = upd_s(part, mxc, invmc, arowc)

    mx3 = mask2f * x2
    invm3 = (1.0 - mask2f) * inv_pre[:, None]
    for t in range(k_low):
        mx3 = mx3.at[:, low_idx[t]].set(outc[:n, t])
        invm3 = invm3.at[:, low_idx[t]].set(0.0)
    mx3 = pad_rows(mx3)
    invm3 = pad_rows(invm3)
    arow3 = pad_rows(jnp.broadcast_to(a2[:, None], (n, d)))
    z3 = arow3 * mx3
    out3 = None
    for _ in range(NUM_ITERATIONS):
        (part,) = scat_b(z3, fsrc, fdst)
        z3, out3 = upd_b(part, mx3, invm3, arow3)

    return out3[:n]
